# Initial kernel scaffold; baseline (speedup 1.0000x reference)
import jax
import jax.numpy as jnp
from jax.experimental import pallas as pl


def kernel(inputs, embeddings):
    raise NotImplementedError("write your pallas kernel here")


# speedup vs baseline: 1.2504x; 1.2504x over previous
import functools

import jax
import jax.numpy as jnp
from jax import lax
from jax.experimental import pallas as pl
from jax.experimental.pallas import tpu as pltpu
from jax.experimental.pallas import tpu_sc as plsc

_R = 8
_K = 5
_TV = 2048
_SLOTS = 8
_NEG = float("-inf")
_BIGI = 2**31 - 1


def _tc_body(x_ref, emb_ref, oidx_ref, xred_ref, rv_ref, ri_ref, *, V):
    B, D = x_ref.shape
    TV = emb_ref.shape[0]
    G = D // _R
    t = pl.program_id(0)

    @pl.when(t == 0)
    def _init():
        x = x_ref[...]
        xred_ref[...] = jnp.sum(x.reshape(B, G, _R), axis=-1)
        rv_ref[...] = jnp.full((B, _SLOTS), _NEG, jnp.float32)
        ri_ref[...] = jnp.zeros((B, _SLOTS), jnp.int32)

    ered = jnp.sum(emb_ref[...].reshape(TV, G, _R), axis=-1)
    s = lax.dot_general(
        xred_ref[...], ered,
        (((1,), (1,)), ((), ())),
        preferred_element_type=jnp.float32,
    )
    col = lax.broadcasted_iota(jnp.int32, (B, TV), 1)
    s = jnp.where(t * TV + col < V, s, _NEG)

    tvs, tis = [], []
    for _ in range(_K):
        m = jnp.max(s, axis=1)
        a = jnp.min(jnp.where(s == m[:, None], col, TV), axis=1)
        tvs.append(m)
        tis.append(t * TV + a)
        s = jnp.where(col == a[:, None], _NEG, s)

    comb_v = jnp.concatenate(
        [rv_ref[...], jnp.stack(tvs, axis=1),
         jnp.full((B, 3), _NEG, jnp.float32)], axis=1)
    comb_i = jnp.concatenate(
        [ri_ref[...], jnp.stack(tis, axis=1),
         jnp.zeros((B, 3), jnp.int32)], axis=1)
    slot = lax.broadcasted_iota(jnp.int32, (B, 16), 1)
    nvs, nis = [], []
    for _ in range(_K):
        m = jnp.max(comb_v, axis=1)
        pos = jnp.min(jnp.where(comb_v == m[:, None], slot, 16), axis=1)
        sel = slot == pos[:, None]
        nvs.append(m)
        nis.append(jnp.min(jnp.where(sel, comb_i, _BIGI), axis=1))
        comb_v = jnp.where(sel, _NEG, comb_v)
    rv_ref[...] = jnp.concatenate(
        [jnp.stack(nvs, axis=1), jnp.full((B, 3), _NEG, jnp.float32)], axis=1)
    ri_ref[...] = jnp.concatenate(
        [jnp.stack(nis, axis=1), jnp.zeros((B, 3), jnp.int32)], axis=1)

    @pl.when(t == pl.num_programs(0) - 1)
    def _emit():
        oidx_ref[...] = ri_ref[...]


def _stage1_topk(inputs, embeddings, interpret=False):
    B, D = inputs.shape
    V = embeddings.shape[0]
    nt = (V + _TV - 1) // _TV
    return pl.pallas_call(
        functools.partial(_tc_body, V=V),
        grid=(nt,),
        in_specs=[
            pl.BlockSpec((B, D), lambda t: (0, 0)),
            pl.BlockSpec((_TV, D), lambda t: (t, 0)),
        ],
        out_specs=pl.BlockSpec((B, _SLOTS), lambda t: (0, 0)),
        out_shape=jax.ShapeDtypeStruct((B, _SLOTS), jnp.int32),
        scratch_shapes=[
            pltpu.VMEM((B, D // _R), jnp.float32),
            pltpu.VMEM((B, _SLOTS), jnp.float32),
            pltpu.VMEM((B, _SLOTS), jnp.int32),
        ],
        interpret=interpret,
    )(inputs, embeddings)


def _sc_body(emb_hbm, x_hbm, idx_hbm, probs_hbm, oidx_hbm,
             idx_a, idx_b, rows_a, rows_b, x_v, probs_v, oidx_v, sem,
             *, D, BPW):
    c = lax.axis_index("c")
    s = lax.axis_index("s")
    wid = s * 2 + c
    base_r = wid * BPW
    base_i = wid * BPW * _SLOTS

    pltpu.sync_copy(idx_hbm.at[pl.ds(base_i, 16 * _SLOTS)], idx_a)
    pltpu.sync_copy(idx_hbm.at[pl.ds(base_i + 16 * _SLOTS, 16 * _SLOTS)], idx_b)
    pltpu.sync_copy(x_hbm.at[pl.ds(base_r, BPW)], x_v)
    cp_a = pltpu.async_copy(emb_hbm.at[idx_a], rows_a, sem)
    cp_b = pltpu.async_copy(emb_hbm.at[idx_b], rows_b, sem)
    cp_a.wait()
    cp_b.wait()

    lane = lax.iota(jnp.int32, 16)
    for g, (rows_g, idx_g) in enumerate(((rows_a, idx_a), (rows_b, idx_b))):
        xrow = g * 16 + lane

        def dot_step(d, accs, rows_g=rows_g, xrow=xrow):
            dv = jnp.full((16,), 0, jnp.int32) + d
            xv = plsc.load_gather(x_v, [xrow, dv])
            return tuple(
                acc + xv * plsc.load_gather(rows_g, [lane * _SLOTS + k, dv])
                for k, acc in enumerate(accs))

        logits = lax.fori_loop(
            0, D, dot_step,
            tuple(jnp.zeros((16,), jnp.float32) for _ in range(_K)))

        m = logits[0]
        for k in range(1, _K):
            m = jnp.maximum(m, logits[k])
        z = jnp.zeros((16,), jnp.float32)
        for k in range(_K):
            z = z + jnp.exp(logits[k] - m)
        prob = 1.0 / z
        best = jnp.full((16,), _K - 1, jnp.int32)
        for k in range(_K - 2, -1, -1):
            best = jnp.where(logits[k] == m, k, best)
        fidx = plsc.load_gather(idx_g, [lane * _SLOTS + best])
        probs_v[pl.ds(g * 16, 16)] = prob
        oidx_v[pl.ds(g * 16, 16)] = fidx

    pltpu.sync_copy(probs_v, probs_hbm.at[pl.ds(base_r, BPW)])
    pltpu.sync_copy(oidx_v, oidx_hbm.at[pl.ds(base_r, BPW)])


def _stage2_rescore(embeddings, inputs, idx_flat):
    B, D = inputs.shape
    BPW = B // 32
    mesh = plsc.VectorSubcoreMesh(core_axis_name="c", subcore_axis_name="s")
    return pl.kernel(
        functools.partial(_sc_body, D=D, BPW=BPW),
        out_type=(
            jax.ShapeDtypeStruct((B,), jnp.float32),
            jax.ShapeDtypeStruct((B,), jnp.int32),
        ),
        mesh=mesh,
        compiler_params=pltpu.CompilerParams(needs_layout_passes=False),
        scratch_types=[
            pltpu.VMEM((16 * _SLOTS,), jnp.int32),
            pltpu.VMEM((16 * _SLOTS,), jnp.int32),
            pltpu.VMEM((16 * _SLOTS, D), jnp.float32),
            pltpu.VMEM((16 * _SLOTS, D), jnp.float32),
            pltpu.VMEM((BPW, D), jnp.float32),
            pltpu.VMEM((BPW,), jnp.float32),
            pltpu.VMEM((BPW,), jnp.int32),
            pltpu.SemaphoreType.DMA,
        ],
    )(embeddings, inputs, idx_flat)


def kernel(inputs, embeddings):
    dims = inputs.shape
    d = dims[-1]
    x = inputs.reshape(-1, d)
    top_idx = _stage1_topk(x, embeddings)
    idx_flat = top_idx.reshape(-1)
    probs, indices = _stage2_rescore(embeddings, x, idx_flat)
    return probs.reshape(dims[:-1]), indices.reshape(dims[:-1])
